# Initial kernel scaffold; baseline (speedup 1.0000x reference)
import jax
import jax.numpy as jnp
from jax.experimental import pallas as pl


def kernel(h, W0, b0, W1, b1):
    raise NotImplementedError("write your pallas kernel here")


# speedup vs baseline: 15.0723x; 15.0723x over previous
import jax
import jax.numpy as jnp
from jax.experimental import pallas as pl

N = 10000
D = 256
KP1 = 21
NEG = -3.0e38


def _bf16_dot_t(a, b):
    return jax.lax.dot_general(
        a.astype(jnp.bfloat16), b.astype(jnp.bfloat16),
        (((1,), (1,)), ((), ())), preferred_element_type=jnp.float32)


def _emb_kernel(h_ref, w0_ref, b0_ref, w1_ref, b1_ref, emb_ref):
    x = jnp.maximum(_bf16_dot_t(h_ref[...], w0_ref[...]) + b0_ref[...], 0.0)
    x = _bf16_dot_t(x, w1_ref[...]) + b1_ref[...]
    n = jnp.sqrt(jnp.sum(x * x, axis=1, keepdims=True))
    emb_ref[...] = x / jnp.maximum(n, 1e-12)


def _topk_kernel(rows_ref, emb_ref, out_ref):
    adj = _bf16_dot_t(rows_ref[...], emb_ref[...])
    m = jnp.max(adj, axis=1, keepdims=True)
    for _ in range(KP1 - 1):
        m = jnp.max(jnp.where(adj < m, adj, NEG), axis=1, keepdims=True)
    out_ref[...] = jnp.where((adj >= m) & (adj > 0.0), adj, 0.0)


def kernel(h, W0, b0, W1, b1):
    b0r = b0.reshape(1, D)
    b1r = b1.reshape(1, D)
    emb = pl.pallas_call(
        _emb_kernel,
        out_shape=jax.ShapeDtypeStruct((N, D), jnp.float32),
    )(h, W0, b0r, W1, b1r)

    R = 200
    out = pl.pallas_call(
        _topk_kernel,
        grid=(N // R,),
        in_specs=[
            pl.BlockSpec((R, D), lambda i: (i, 0)),
            pl.BlockSpec((N, D), lambda i: (0, 0)),
        ],
        out_specs=pl.BlockSpec((R, N), lambda i: (i, 0)),
        out_shape=jax.ShapeDtypeStruct((N, N), jnp.float32),
    )(emb, emb)
    return out


# speedup vs baseline: 28.4420x; 1.8870x over previous
import jax
import jax.numpy as jnp
from jax.experimental import pallas as pl
from jax.experimental.pallas import tpu as pltpu

N = 10000
D = 256
KP1 = 21
NEG = -3.0e38
CH = (N + 127) // 128
NP = CH * 128
L = 4


def _bf16_dot_t(a, b):
    return jax.lax.dot_general(
        a.astype(jnp.bfloat16), b.astype(jnp.bfloat16),
        (((1,), (1,)), ((), ())), preferred_element_type=jnp.float32)


def _emb_kernel(h_ref, w0_ref, b0_ref, w1_ref, b1_ref, emb_ref):
    x = jnp.maximum(_bf16_dot_t(h_ref[...], w0_ref[...]) + b0_ref[...], 0.0)
    x = _bf16_dot_t(x, w1_ref[...]) + b1_ref[...]
    n = jnp.sqrt(jnp.sum(x * x, axis=1, keepdims=True))
    emb_ref[...] = x / jnp.maximum(n, 1e-12)


def _topk_kernel(rows_ref, emb_ref, out_ref, thr_ref):
    R = rows_ref.shape[0]
    adj = _bf16_dot_t(rows_ref[...], emb_ref[...])
    col = jax.lax.broadcasted_iota(jnp.int32, (R, NP), 1)
    adjm = jnp.where(col < N, adj, NEG)

    tops = []
    prev = None
    for _ in range(L):
        acc = jnp.full((R, 128), NEG, jnp.float32)
        for c in range(CH):
            s = adjm[:, c * 128:(c + 1) * 128]
            if prev is not None:
                s = jnp.where(s < prev, s, NEG)
            acc = jnp.maximum(acc, s)
        tops.append(acc)
        prev = acc

    m = jnp.max(tops[0], axis=1, keepdims=True)
    for _ in range(KP1 - 1):
        acc = jnp.full((R, 128), NEG, jnp.float32)
        for t in tops:
            acc = jnp.maximum(acc, jnp.where(t < m, t, NEG))
        m = jnp.max(acc, axis=1, keepdims=True)

    ge = jnp.sum(jnp.where(adjm >= m, 1.0, 0.0), axis=1, keepdims=True)
    gt = jnp.sum(jnp.where(adjm > m, 1.0, 0.0), axis=1, keepdims=True)
    bad = jnp.any((ge < float(KP1)) | (gt > float(KP1 - 1)))

    thr_ref[...] = jnp.broadcast_to(m, (R, 128))

    @pl.when(bad)
    def _fallback():
        mm = jnp.max(adjm, axis=1, keepdims=True)
        for _ in range(KP1 - 1):
            mm = jnp.max(jnp.where(adjm < mm, adjm, NEG), axis=1,
                         keepdims=True)
        thr_ref[...] = jnp.broadcast_to(mm, (R, 128))

    thr = thr_ref[:, 0:1]
    out_ref[...] = jnp.where((adjm >= thr) & (adjm > 0.0), adjm, 0.0)[:, :N]


def kernel(h, W0, b0, W1, b1):
    b0r = b0.reshape(1, D)
    b1r = b1.reshape(1, D)
    emb = pl.pallas_call(
        _emb_kernel,
        out_shape=jax.ShapeDtypeStruct((N, D), jnp.float32),
    )(h, W0, b0r, W1, b1r)

    embp = jnp.concatenate(
        [emb, jnp.zeros((NP - N, D), jnp.float32)], axis=0)

    R = 200
    out = pl.pallas_call(
        _topk_kernel,
        grid=(N // R,),
        in_specs=[
            pl.BlockSpec((R, D), lambda i: (i, 0)),
            pl.BlockSpec((NP, D), lambda i: (0, 0)),
        ],
        out_specs=pl.BlockSpec((R, N), lambda i: (i, 0)),
        out_shape=jax.ShapeDtypeStruct((N, N), jnp.float32),
        scratch_shapes=[pltpu.VMEM((R, 128), jnp.float32)],
    )(emb, embp)
    return out


# speedup vs baseline: 35.5493x; 1.2499x over previous
import jax
import jax.numpy as jnp
from jax.experimental import pallas as pl

N = 10000
D = 256
KP1 = 21
NEG = -3.0e38
CH = (N + 127) // 128
NP = CH * 128
TAIL = N - (CH - 1) * 128
L = 4


def _bf16_dot_t(a, b):
    return jax.lax.dot_general(
        a.astype(jnp.bfloat16), b.astype(jnp.bfloat16),
        (((1,), (1,)), ((), ())), preferred_element_type=jnp.float32)


def _emb_kernel(h_ref, w0_ref, b0_ref, w1_ref, b1_ref, emb_ref):
    x = jnp.maximum(_bf16_dot_t(h_ref[...], w0_ref[...]) + b0_ref[...], 0.0)
    x = _bf16_dot_t(x, w1_ref[...]) + b1_ref[...]
    n = jnp.sqrt(jnp.sum(x * x, axis=1, keepdims=True))
    emb_ref[...] = x / jnp.maximum(n, 1e-12)


def _topk_kernel(rows_ref, emb_ref, out_ref):
    R = rows_ref.shape[0]
    adj = _bf16_dot_t(rows_ref[...], emb_ref[...])
    lane = jax.lax.broadcasted_iota(jnp.int32, (R, 128), 1)
    tailmask = lane < TAIL

    M = [jnp.full((R, 128), NEG, jnp.float32) for _ in range(L)]
    for c in range(CH):
        s = adj[:, c * 128:(c + 1) * 128]
        if c == CH - 1:
            s = jnp.where(tailmask, s, NEG)
        for i in range(L):
            hi = jnp.maximum(M[i], s)
            s = jnp.minimum(M[i], s)
            M[i] = hi

    m = jnp.max(M[0], axis=1, keepdims=True)
    for _ in range(KP1 - 1):
        acc = jnp.full((R, 128), NEG, jnp.float32)
        for t in M:
            acc = jnp.maximum(acc, jnp.where(t < m, t, NEG))
        m = jnp.max(acc, axis=1, keepdims=True)

    gtacc = jnp.zeros((R, 128), jnp.float32)
    for c in range(CH):
        s = adj[:, c * 128:(c + 1) * 128]
        if c == CH - 1:
            s = jnp.where(tailmask, s, NEG)
        gtacc += jnp.where(s > m, 1.0, 0.0)
        o = jnp.where((s >= m) & (s > 0.0), s, 0.0)
        if c == CH - 1:
            out_ref[:, c * 128:N] = o[:, :TAIL]
        else:
            out_ref[:, c * 128:(c + 1) * 128] = o
    bad = jnp.any(jnp.sum(gtacc, axis=1) > float(KP1 - 1))

    @pl.when(bad)
    def _fallback():
        col = jax.lax.broadcasted_iota(jnp.int32, (R, NP), 1)
        adjm = jnp.where(col < N, adj, NEG)
        mm = jnp.max(adjm, axis=1, keepdims=True)
        for _ in range(KP1 - 1):
            mm = jnp.max(jnp.where(adjm < mm, adjm, NEG), axis=1,
                         keepdims=True)
        out_ref[...] = jnp.where((adjm >= mm) & (adjm > 0.0), adjm,
                                 0.0)[:, :N]


def kernel(h, W0, b0, W1, b1):
    b0r = b0.reshape(1, D)
    b1r = b1.reshape(1, D)
    emb = pl.pallas_call(
        _emb_kernel,
        out_shape=jax.ShapeDtypeStruct((N, D), jnp.float32),
    )(h, W0, b0r, W1, b1r)

    embp = jnp.concatenate(
        [emb, jnp.zeros((NP - N, D), jnp.float32)], axis=0)

    R = 200
    out = pl.pallas_call(
        _topk_kernel,
        grid=(N // R,),
        in_specs=[
            pl.BlockSpec((R, D), lambda i: (i, 0)),
            pl.BlockSpec((NP, D), lambda i: (0, 0)),
        ],
        out_specs=pl.BlockSpec((R, N), lambda i: (i, 0)),
        out_shape=jax.ShapeDtypeStruct((N, N), jnp.float32),
    )(emb, embp)
    return out


# speedup vs baseline: 41.6830x; 1.1725x over previous
import jax
import jax.numpy as jnp
from jax.experimental import pallas as pl

N = 10000
D = 256
KP1 = 21
NEG = -3.0e38
CH = (N + 127) // 128
NP = CH * 128
TAIL = N - (CH - 1) * 128
L = 4


def _bf16_dot_t(a, b):
    return jax.lax.dot_general(
        a.astype(jnp.bfloat16), b.astype(jnp.bfloat16),
        (((1,), (1,)), ((), ())), preferred_element_type=jnp.float32)


def _emb_kernel(h_ref, w0_ref, b0_ref, w1_ref, b1_ref, emb_ref):
    x = jnp.maximum(_bf16_dot_t(h_ref[...], w0_ref[...]) + b0_ref[...], 0.0)
    x = _bf16_dot_t(x, w1_ref[...]) + b1_ref[...]
    n = jnp.sqrt(jnp.sum(x * x, axis=1, keepdims=True))
    emb_ref[:N, :] = x / jnp.maximum(n, 1e-12)
    emb_ref[N:, :] = jnp.zeros((NP - N, D), jnp.float32)


def _topk_kernel(rows_ref, emb_ref, out_ref):
    R = rows_ref.shape[0]
    adj = _bf16_dot_t(rows_ref[...], emb_ref[...])
    lane = jax.lax.broadcasted_iota(jnp.int32, (R, 128), 1)
    tailmask = lane < TAIL

    def _ce(a, b):
        return jnp.maximum(a, b), jnp.minimum(a, b)

    def _sl(c):
        s = adj[:, c * 128:(c + 1) * 128]
        if c == CH - 1:
            s = jnp.where(tailmask, s, NEG)
        return s

    ngroups = CH // 4
    M = None
    for g in range(ngroups):
        s0 = _sl(4 * g + 0)
        s1 = _sl(4 * g + 1)
        s2 = _sl(4 * g + 2)
        s3 = _sl(4 * g + 3)
        hi1, lo1 = _ce(s0, s1)
        hi2, lo2 = _ce(s2, s3)
        b1, mid1 = _ce(hi1, hi2)
        mid2, b4 = _ce(lo1, lo2)
        b2, b3 = _ce(mid1, mid2)
        if M is None:
            M = [b1, b2, b3, b4]
        else:
            c1 = jnp.maximum(M[0], b4)
            c2 = jnp.maximum(M[1], b3)
            c3 = jnp.maximum(M[2], b2)
            c4 = jnp.maximum(M[3], b1)
            x1, x3 = _ce(c1, c3)
            x2, x4 = _ce(c2, c4)
            a1, a2 = _ce(x1, x2)
            a3, a4 = _ce(x3, x4)
            M = [a1, a2, a3, a4]
    for c in range(4 * ngroups, CH):
        s = _sl(c)
        for i in range(L):
            hi = jnp.maximum(M[i], s)
            s = jnp.minimum(M[i], s)
            M[i] = hi

    m = jnp.max(M[0], axis=1, keepdims=True)
    for _ in range(KP1 - 1):
        w = [jnp.where(t < m, t, NEG) for t in M]
        acc = jnp.maximum(jnp.maximum(w[0], w[1]),
                          jnp.maximum(w[2], w[3]))
        m = jnp.max(acc, axis=1, keepdims=True)

    thr = jnp.maximum(m, 1e-38)
    geacc = jnp.zeros((R, 128), jnp.float32)
    for c in range(CH):
        s = _sl(c)
        keep = s >= thr
        geacc += jnp.where(keep, 1.0, 0.0)
        o = jnp.where(keep, s, 0.0)
        if c == CH - 1:
            out_ref[:, c * 128:N] = o[:, :TAIL]
        else:
            out_ref[:, c * 128:(c + 1) * 128] = o
    bad = jnp.any((jnp.sum(geacc, axis=1, keepdims=True) > float(KP1)) |
                  (m < 1e-38))

    @pl.when(bad)
    def _fallback():
        col = jax.lax.broadcasted_iota(jnp.int32, (R, NP), 1)
        adjm = jnp.where(col < N, adj, NEG)
        mm = jnp.max(adjm, axis=1, keepdims=True)
        for _ in range(KP1 - 1):
            mm = jnp.max(jnp.where(adjm < mm, adjm, NEG), axis=1,
                         keepdims=True)
        out_ref[...] = jnp.where((adjm >= mm) & (adjm > 0.0), adjm,
                                 0.0)[:, :N]


def kernel(h, W0, b0, W1, b1):
    b0r = b0.reshape(1, D)
    b1r = b1.reshape(1, D)
    embp = pl.pallas_call(
        _emb_kernel,
        out_shape=jax.ShapeDtypeStruct((NP, D), jnp.float32),
    )(h, W0, b0r, W1, b1r)

    R = 200
    out = pl.pallas_call(
        _topk_kernel,
        grid=(N // R,),
        in_specs=[
            pl.BlockSpec((R, D), lambda i: (i, 0)),
            pl.BlockSpec((NP, D), lambda i: (0, 0)),
        ],
        out_specs=pl.BlockSpec((R, N), lambda i: (i, 0)),
        out_shape=jax.ShapeDtypeStruct((N, N), jnp.float32),
    )(embp, embp)
    return out


# speedup vs baseline: 41.9316x; 1.0060x over previous
import jax
import jax.numpy as jnp
from jax.experimental import pallas as pl

N = 10000
D = 256
KP1 = 21
NEG = -3.0e38
CH = (N + 127) // 128
NP = CH * 128
TAIL = N - (CH - 1) * 128
L = 4


def _bf16_dot_t(a, b):
    return jax.lax.dot_general(
        a.astype(jnp.bfloat16), b.astype(jnp.bfloat16),
        (((1,), (1,)), ((), ())), preferred_element_type=jnp.float32)


def _emb_kernel(h_ref, w0_ref, b0_ref, w1_ref, b1_ref, emb_ref):
    x = jnp.maximum(_bf16_dot_t(h_ref[...], w0_ref[...]) + b0_ref[...], 0.0)
    x = _bf16_dot_t(x, w1_ref[...]) + b1_ref[...]
    n = jnp.sqrt(jnp.sum(x * x, axis=1, keepdims=True))
    emb_ref[:N, :] = x / jnp.maximum(n, 1e-12)
    emb_ref[N:, :] = jnp.zeros((NP - N, D), jnp.float32)


def _topk_kernel(rows_ref, emb_ref, out_ref):
    R = rows_ref.shape[0]
    adj = _bf16_dot_t(rows_ref[...], emb_ref[...])
    lane = jax.lax.broadcasted_iota(jnp.int32, (R, 128), 1)
    tailmask = lane < TAIL

    def _ce(a, b):
        return jnp.maximum(a, b), jnp.minimum(a, b)

    def _sl(c):
        s = adj[:, c * 128:(c + 1) * 128]
        if c == CH - 1:
            s = jnp.where(tailmask, s, NEG)
        return s

    ngroups = CH // 4
    M = None
    for g in range(ngroups):
        s0 = _sl(4 * g + 0)
        s1 = _sl(4 * g + 1)
        s2 = _sl(4 * g + 2)
        s3 = _sl(4 * g + 3)
        hi1, lo1 = _ce(s0, s1)
        hi2, lo2 = _ce(s2, s3)
        b1, mid1 = _ce(hi1, hi2)
        mid2, b4 = _ce(lo1, lo2)
        b2, b3 = _ce(mid1, mid2)
        if M is None:
            M = [b1, b2, b3, b4]
        else:
            c1 = jnp.maximum(M[0], b4)
            c2 = jnp.maximum(M[1], b3)
            c3 = jnp.maximum(M[2], b2)
            c4 = jnp.maximum(M[3], b1)
            x1, x3 = _ce(c1, c3)
            x2, x4 = _ce(c2, c4)
            a1, a2 = _ce(x1, x2)
            a3, a4 = _ce(x3, x4)
            M = [a1, a2, a3, a4]
    for c in range(4 * ngroups, CH):
        s = _sl(c)
        for i in range(L):
            hi = jnp.maximum(M[i], s)
            s = jnp.minimum(M[i], s)
            M[i] = hi

    p1, p2, p3, p4 = M
    m = jnp.max(p1, axis=1, keepdims=True)
    for _ in range(KP1 - 1):
        hit = p1 == m
        p1 = jnp.where(hit, p2, p1)
        p2 = jnp.where(hit, p3, p2)
        p3 = jnp.where(hit, p4, p3)
        p4 = jnp.where(hit, NEG, p4)
        m = jnp.max(p1, axis=1, keepdims=True)

    thr = jnp.maximum(m, 1e-38)
    geacc = jnp.zeros((R, 128), jnp.float32)
    for c in range(CH):
        s = _sl(c)
        keep = s >= thr
        geacc += jnp.where(keep, 1.0, 0.0)
        o = jnp.where(keep, s, 0.0)
        if c == CH - 1:
            out_ref[:, c * 128:N] = o[:, :TAIL]
        else:
            out_ref[:, c * 128:(c + 1) * 128] = o
    bad = jnp.any((jnp.sum(geacc, axis=1, keepdims=True) > float(KP1)) |
                  (m < 1e-38))

    @pl.when(bad)
    def _fallback():
        col = jax.lax.broadcasted_iota(jnp.int32, (R, NP), 1)
        adjm = jnp.where(col < N, adj, NEG)
        mm = jnp.max(adjm, axis=1, keepdims=True)
        for _ in range(KP1 - 1):
            mm = jnp.max(jnp.where(adjm < mm, adjm, NEG), axis=1,
                         keepdims=True)
        out_ref[...] = jnp.where((adjm >= mm) & (adjm > 0.0), adjm,
                                 0.0)[:, :N]


def kernel(h, W0, b0, W1, b1):
    b0r = b0.reshape(1, D)
    b1r = b1.reshape(1, D)
    embp = pl.pallas_call(
        _emb_kernel,
        out_shape=jax.ShapeDtypeStruct((NP, D), jnp.float32),
    )(h, W0, b0r, W1, b1r)

    R = 200
    out = pl.pallas_call(
        _topk_kernel,
        grid=(N // R,),
        in_specs=[
            pl.BlockSpec((R, D), lambda i: (i, 0)),
            pl.BlockSpec((NP, D), lambda i: (0, 0)),
        ],
        out_specs=pl.BlockSpec((R, N), lambda i: (i, 0)),
        out_shape=jax.ShapeDtypeStruct((N, N), jnp.float32),
    )(embp, embp)
    return out
